# Initial kernel scaffold; baseline (speedup 1.0000x reference)
import jax
import jax.numpy as jnp
from jax.experimental import pallas as pl


def kernel(idxs, values, feats, candidates, scale, bias, which_axis):
    raise NotImplementedError("write your pallas kernel here")


# speedup vs baseline: 6.0626x; 6.0626x over previous
import jax
import jax.numpy as jnp
from jax import lax
from jax.experimental import pallas as pl
from jax.experimental.pallas import tpu as pltpu
from jax.experimental.pallas import tpu_sc as plsc

D0 = 4096
SUMD = 2 * D0
F = 256
H = 128
NCAND = 16
BQ = 32768
L = 16
NC, NS = 2, 16
NW = NC * NS
K = 64
RB = 1024


def _prep_body(feats_ref, cand_ref, values_ref, m_ref, aux_ref):
    i = pl.program_id(0)
    is0 = i < (pl.num_programs(0) // 2)
    f = feats_ref[...]
    cand = cand_ref[0]
    vals = values_ref[0]
    c0v = cand[:, 0]
    best = jnp.abs(f - c0v)
    q = jnp.broadcast_to(c0v, f.shape)
    for k in range(1, NCAND):
        ck = cand[:, k]
        d = jnp.abs(f - ck)
        better = d < best
        q = jnp.where(better, ck, q)
        best = jnp.where(better, d, best)
    w1 = vals[1] - vals[3]
    w2 = vals[2] - vals[3]
    w3 = vals[0] - vals[1] - vals[2] + vals[3]
    wa = jnp.where(is0, w1, w2)
    a0 = jnp.sum(q[:, :H] * wa[:H], axis=1)
    a1 = jnp.sum(q[:, H:] * wa[H:], axis=1)
    m_ref[...] = q * jnp.where(is0, w3, jnp.ones_like(w3))
    aux_ref[...] = jnp.stack([a0, a1], axis=0)


def _prep(feats, candidates, values):
    grid = SUMD // RB
    return pl.pallas_call(
        _prep_body,
        grid=(grid,),
        in_specs=[
            pl.BlockSpec((RB, F), lambda i: (i, 0)),
            pl.BlockSpec((1, F, NCAND), lambda i: (i // (SUMD // RB // 2), 0, 0)),
            pl.BlockSpec((1, 4, F), lambda i: (0, 0, 0)),
        ],
        out_specs=[
            pl.BlockSpec((RB, F), lambda i: (i, 0)),
            pl.BlockSpec((2, RB), lambda i: (0, i)),
        ],
        out_shape=[
            jax.ShapeDtypeStruct((SUMD, F), jnp.float32),
            jax.ShapeDtypeStruct((2, SUMD), jnp.float32),
        ],
    )(feats, candidates, values)


def _sc_body(m_hbm, aux_hbm, i0_hbm, i1_hbm, out0_hbm, out1_hbm,
             aux0_v, aux1_v, idx0_v, idx1_v, r0_v, r1_v, s0_v, s1_v,
             p0_v, p1_v, sem0, sem1):
    wid = lax.axis_index("s") * NC + lax.axis_index("c")
    per_w = BQ // NW
    nchunk = per_w // K
    lanes = lax.broadcasted_iota(jnp.int32, (L,), 0)

    pltpu.sync_copy(aux_hbm.at[0], aux0_v)
    pltpu.sync_copy(aux_hbm.at[1], aux1_v)

    def chunk(ci, carry):
        base = wid * per_w + ci * K
        pltpu.sync_copy(i0_hbm.at[pl.ds(base, K)], idx0_v)
        pltpu.sync_copy(i1_hbm.at[pl.ds(base, K)], idx1_v)
        cp0 = pltpu.async_copy(m_hbm.at[idx0_v], r0_v, sem0)
        cp1 = pltpu.async_copy(m_hbm.at[idx1_v], r1_v, sem1)
        cp0.wait()
        cp1.wait()
        for j0 in range(0, K, L):
            for jj in range(L):
                j = j0 + jj
                acc0 = r0_v[j, pl.ds(0, L)] * r1_v[j, pl.ds(0, L)]
                acc1 = r0_v[j, pl.ds(8 * L, L)] * r1_v[j, pl.ds(8 * L, L)]
                for t in range(1, 8):
                    acc0 = acc0 + r0_v[j, pl.ds(t * L, L)] * r1_v[j, pl.ds(t * L, L)]
                for t in range(9, 16):
                    acc1 = acc1 + r0_v[j, pl.ds(t * L, L)] * r1_v[j, pl.ds(t * L, L)]
                p0_v[jj, :] = acc0
                p1_v[jj, :] = acc1
            o0 = jnp.zeros((L,), jnp.float32)
            o1 = jnp.zeros((L,), jnp.float32)
            for l in range(L):
                col = jnp.full((L,), l, jnp.int32)
                o0 = o0 + plsc.load_gather(p0_v, [lanes, col])
                o1 = o1 + plsc.load_gather(p1_v, [lanes, col])
            i0reg = idx0_v[pl.ds(j0, L)]
            i1reg = idx1_v[pl.ds(j0, L)]
            o0 = o0 + plsc.load_gather(aux0_v, [i0reg]) + plsc.load_gather(aux0_v, [i1reg])
            o1 = o1 + plsc.load_gather(aux1_v, [i0reg]) + plsc.load_gather(aux1_v, [i1reg])
            s0_v[pl.ds(j0, L)] = o0
            s1_v[pl.ds(j0, L)] = o1
        pltpu.sync_copy(s0_v, out0_hbm.at[pl.ds(base, K)])
        pltpu.sync_copy(s1_v, out1_hbm.at[pl.ds(base, K)])
        return carry

    lax.fori_loop(0, nchunk, chunk, 0)


def _sc_contract(m, aux, i0, i1):
    return pl.kernel(
        _sc_body,
        out_type=(
            jax.ShapeDtypeStruct((BQ,), jnp.float32),
            jax.ShapeDtypeStruct((BQ,), jnp.float32),
        ),
        mesh=plsc.VectorSubcoreMesh(core_axis_name="c", subcore_axis_name="s"),
        compiler_params=pltpu.CompilerParams(needs_layout_passes=False),
        scratch_types=[
            pltpu.VMEM((SUMD,), jnp.float32),
            pltpu.VMEM((SUMD,), jnp.float32),
            pltpu.VMEM((K,), jnp.int32),
            pltpu.VMEM((K,), jnp.int32),
            pltpu.VMEM((K, F), jnp.float32),
            pltpu.VMEM((K, F), jnp.float32),
            pltpu.VMEM((K,), jnp.float32),
            pltpu.VMEM((K,), jnp.float32),
            pltpu.VMEM((L, L), jnp.float32),
            pltpu.VMEM((L, L), jnp.float32),
            pltpu.SemaphoreType.DMA,
            pltpu.SemaphoreType.DMA,
        ],
    )(m, aux, i0, i1)


def _fin_body(s0_ref, s1_ref, values_ref, scb_ref, out_ref):
    base = values_ref[0, 3]
    c0 = jnp.sum(base[:H])
    c1 = jnp.sum(base[H:])
    esc = jnp.exp(scb_ref[0, 0])
    b = scb_ref[0, 1]
    out_ref[...] = (s0_ref[...] + c0) * jnp.tanh(s1_ref[...] + c1) * esc + b


def _finalize(s0, s1, values, scale, bias):
    scb = jnp.concatenate([scale, bias]).reshape(1, 2)
    out = pl.pallas_call(
        _fin_body,
        in_specs=[
            pl.BlockSpec((BQ // H, H), lambda: (0, 0)),
            pl.BlockSpec((BQ // H, H), lambda: (0, 0)),
            pl.BlockSpec((1, 4, F), lambda: (0, 0, 0)),
            pl.BlockSpec(memory_space=pltpu.SMEM),
        ],
        out_specs=pl.BlockSpec((BQ // H, H), lambda: (0, 0)),
        out_shape=jax.ShapeDtypeStruct((BQ // H, H), jnp.float32),
    )(s0.reshape(BQ // H, H), s1.reshape(BQ // H, H), values, scb)
    return out.reshape(BQ)


def kernel(idxs, values, feats, candidates, scale, bias, which_axis):
    i0 = idxs[0].astype(jnp.int32)
    i1 = idxs[1].astype(jnp.int32) + D0
    m, aux = _prep(feats, candidates, values)
    s0, s1 = _sc_contract(m, aux, i0, i1)
    return _finalize(s0, s1, values, scale, bias)


# speedup vs baseline: 11.1320x; 1.8362x over previous
import jax
import jax.numpy as jnp
from jax import lax
from jax.experimental import pallas as pl
from jax.experimental.pallas import tpu as pltpu
from jax.experimental.pallas import tpu_sc as plsc

D0 = 4096
SUMD = 2 * D0
F = 256
H = 128
NCAND = 16
BQ = 32768
L = 16
NC, NS = 2, 16
NW = NC * NS
K = 64
RB = 1024


def _prep_body(feats_ref, cand_ref, values_ref, m_ref, aux_ref):
    i = pl.program_id(0)
    is0 = i < (pl.num_programs(0) // 2)
    f = feats_ref[...]
    cand = cand_ref[0]
    vals = values_ref[0]
    c0v = cand[:, 0]
    best = jnp.abs(f - c0v)
    q = jnp.broadcast_to(c0v, f.shape)
    for k in range(1, NCAND):
        ck = cand[:, k]
        d = jnp.abs(f - ck)
        better = d < best
        q = jnp.where(better, ck, q)
        best = jnp.where(better, d, best)
    w1 = vals[1] - vals[3]
    w2 = vals[2] - vals[3]
    w3 = vals[0] - vals[1] - vals[2] + vals[3]
    wa = jnp.where(is0, w1, w2)
    a0 = jnp.sum(q[:, :H] * wa[:H], axis=1)
    a1 = jnp.sum(q[:, H:] * wa[H:], axis=1)
    m_ref[...] = q * jnp.where(is0, w3, jnp.ones_like(w3))
    aux_ref[...] = jnp.stack([a0, a1], axis=0)


def _prep(feats, candidates, values):
    grid = SUMD // RB
    return pl.pallas_call(
        _prep_body,
        grid=(grid,),
        in_specs=[
            pl.BlockSpec((RB, F), lambda i: (i, 0)),
            pl.BlockSpec((1, F, NCAND), lambda i: (i // (SUMD // RB // 2), 0, 0)),
            pl.BlockSpec((1, 4, F), lambda i: (0, 0, 0)),
        ],
        out_specs=[
            pl.BlockSpec((RB, F), lambda i: (i, 0)),
            pl.BlockSpec((2, RB), lambda i: (0, i)),
        ],
        out_shape=[
            jax.ShapeDtypeStruct((SUMD, F), jnp.float32),
            jax.ShapeDtypeStruct((2, SUMD), jnp.float32),
        ],
    )(feats, candidates, values)


def _sc_body(m_hbm, aux_hbm, i0_hbm, i1_hbm, out0_hbm, out1_hbm,
             aux0_v, aux1_v,
             ia0_v, ia1_v, ib0_v, ib1_v,
             ra0_v, ra1_v, rb0_v, rb1_v,
             s0_v, s1_v, p0_v, p1_v,
             semA0, semA1, semB0, semB1):
    wid = lax.axis_index("s") * NC + lax.axis_index("c")
    per_w = BQ // NW
    nchunk = per_w // K
    w_base = wid * per_w
    lanes = lax.broadcasted_iota(jnp.int32, (L,), 0)
    last = nchunk - 1

    pltpu.sync_copy(aux_hbm.at[0], aux0_v)
    pltpu.sync_copy(aux_hbm.at[1], aux1_v)

    def load_idx(ci, i0_v, i1_v):
        base = w_base + ci * K
        pltpu.sync_copy(i0_hbm.at[pl.ds(base, K)], i0_v)
        pltpu.sync_copy(i1_hbm.at[pl.ds(base, K)], i1_v)

    def fire(i0_v, i1_v, r0_v, r1_v, sem0, sem1):
        pltpu.make_async_copy(m_hbm.at[i0_v], r0_v, sem0).start()
        pltpu.make_async_copy(m_hbm.at[i1_v], r1_v, sem1).start()

    def drain(i0_v, i1_v, r0_v, r1_v, sem0, sem1):
        pltpu.make_async_copy(m_hbm.at[i0_v], r0_v, sem0).wait()
        pltpu.make_async_copy(m_hbm.at[i1_v], r1_v, sem1).wait()

    def compute(ci, i0_v, i1_v, r0_v, r1_v):
        def group(gi, carry):
            j0 = gi * L
            for jj in range(L):
                acc0 = r0_v[j0 + jj, pl.ds(0, L)] * r1_v[j0 + jj, pl.ds(0, L)]
                acc1 = r0_v[j0 + jj, pl.ds(8 * L, L)] * r1_v[j0 + jj, pl.ds(8 * L, L)]
                for t in range(1, 8):
                    acc0 = acc0 + r0_v[j0 + jj, pl.ds(t * L, L)] * r1_v[j0 + jj, pl.ds(t * L, L)]
                for t in range(9, 16):
                    acc1 = acc1 + r0_v[j0 + jj, pl.ds(t * L, L)] * r1_v[j0 + jj, pl.ds(t * L, L)]
                p0_v[jj, :] = acc0
                p1_v[jj, :] = acc1
            o0 = jnp.zeros((L,), jnp.float32)
            o1 = jnp.zeros((L,), jnp.float32)
            for l in range(L):
                col = jnp.full((L,), l, jnp.int32)
                o0 = o0 + plsc.load_gather(p0_v, [lanes, col])
                o1 = o1 + plsc.load_gather(p1_v, [lanes, col])
            i0reg = i0_v[pl.ds(j0, L)]
            i1reg = i1_v[pl.ds(j0, L)]
            o0 = o0 + plsc.load_gather(aux0_v, [i0reg]) + plsc.load_gather(aux0_v, [i1reg])
            o1 = o1 + plsc.load_gather(aux1_v, [i0reg]) + plsc.load_gather(aux1_v, [i1reg])
            s0_v[pl.ds(j0, L)] = o0
            s1_v[pl.ds(j0, L)] = o1
            return carry

        lax.fori_loop(0, K // L, group, 0)
        base = w_base + ci * K
        pltpu.sync_copy(s0_v, out0_hbm.at[pl.ds(base, K)])
        pltpu.sync_copy(s1_v, out1_hbm.at[pl.ds(base, K)])

    load_idx(0, ia0_v, ia1_v)
    fire(ia0_v, ia1_v, ra0_v, ra1_v, semA0, semA1)
    load_idx(1, ib0_v, ib1_v)
    fire(ib0_v, ib1_v, rb0_v, rb1_v, semB0, semB1)

    def pair(p, carry):
        c = 2 * p
        drain(ia0_v, ia1_v, ra0_v, ra1_v, semA0, semA1)
        compute(c, ia0_v, ia1_v, ra0_v, ra1_v)
        cn = jnp.minimum(c + 2, last)
        load_idx(cn, ia0_v, ia1_v)
        fire(ia0_v, ia1_v, ra0_v, ra1_v, semA0, semA1)
        drain(ib0_v, ib1_v, rb0_v, rb1_v, semB0, semB1)
        compute(c + 1, ib0_v, ib1_v, rb0_v, rb1_v)
        cn = jnp.minimum(c + 3, last)
        load_idx(cn, ib0_v, ib1_v)
        fire(ib0_v, ib1_v, rb0_v, rb1_v, semB0, semB1)
        return carry

    lax.fori_loop(0, nchunk // 2, pair, 0)
    drain(ia0_v, ia1_v, ra0_v, ra1_v, semA0, semA1)
    drain(ib0_v, ib1_v, rb0_v, rb1_v, semB0, semB1)


def _sc_contract(m, aux, i0, i1):
    return pl.kernel(
        _sc_body,
        out_type=(
            jax.ShapeDtypeStruct((BQ,), jnp.float32),
            jax.ShapeDtypeStruct((BQ,), jnp.float32),
        ),
        mesh=plsc.VectorSubcoreMesh(core_axis_name="c", subcore_axis_name="s"),
        compiler_params=pltpu.CompilerParams(needs_layout_passes=False),
        scratch_types=[
            pltpu.VMEM((SUMD,), jnp.float32),
            pltpu.VMEM((SUMD,), jnp.float32),
            pltpu.VMEM((K,), jnp.int32),
            pltpu.VMEM((K,), jnp.int32),
            pltpu.VMEM((K,), jnp.int32),
            pltpu.VMEM((K,), jnp.int32),
            pltpu.VMEM((K, F), jnp.float32),
            pltpu.VMEM((K, F), jnp.float32),
            pltpu.VMEM((K, F), jnp.float32),
            pltpu.VMEM((K, F), jnp.float32),
            pltpu.VMEM((K,), jnp.float32),
            pltpu.VMEM((K,), jnp.float32),
            pltpu.VMEM((L, L), jnp.float32),
            pltpu.VMEM((L, L), jnp.float32),
            pltpu.SemaphoreType.DMA,
            pltpu.SemaphoreType.DMA,
            pltpu.SemaphoreType.DMA,
            pltpu.SemaphoreType.DMA,
        ],
    )(m, aux, i0, i1)


def _fin_body(s0_ref, s1_ref, values_ref, scb_ref, out_ref):
    base = values_ref[0, 3]
    c0 = jnp.sum(base[:H])
    c1 = jnp.sum(base[H:])
    esc = jnp.exp(scb_ref[0, 0])
    b = scb_ref[0, 1]
    out_ref[...] = (s0_ref[...] + c0) * jnp.tanh(s1_ref[...] + c1) * esc + b


def _finalize(s0, s1, values, scale, bias):
    scb = jnp.concatenate([scale, bias]).reshape(1, 2)
    out = pl.pallas_call(
        _fin_body,
        in_specs=[
            pl.BlockSpec((BQ // H, H), lambda: (0, 0)),
            pl.BlockSpec((BQ // H, H), lambda: (0, 0)),
            pl.BlockSpec((1, 4, F), lambda: (0, 0, 0)),
            pl.BlockSpec(memory_space=pltpu.SMEM),
        ],
        out_specs=pl.BlockSpec((BQ // H, H), lambda: (0, 0)),
        out_shape=jax.ShapeDtypeStruct((BQ // H, H), jnp.float32),
    )(s0.reshape(BQ // H, H), s1.reshape(BQ // H, H), values, scb)
    return out.reshape(BQ)


def kernel(idxs, values, feats, candidates, scale, bias, which_axis):
    i0 = idxs[0].astype(jnp.int32)
    i1 = idxs[1].astype(jnp.int32) + D0
    m, aux = _prep(feats, candidates, values)
    s0, s1 = _sc_contract(m, aux, i0, i1)
    return _finalize(s0, s1, values, scale, bias)


# speedup vs baseline: 11.6114x; 1.0431x over previous
import jax
import jax.numpy as jnp
from jax import lax
from jax.experimental import pallas as pl
from jax.experimental.pallas import tpu as pltpu
from jax.experimental.pallas import tpu_sc as plsc

D0 = 4096
SUMD = 2 * D0
F = 256
H = 128
NCAND = 16
BQ = 32768
L = 16
NC, NS = 2, 16
NW = NC * NS
K = 64
RB = 1024


def _prep_body(feats_ref, cand_ref, values_ref, scale_ref, bias_ref,
               m_ref, aux_ref, params_ref):
    i = pl.program_id(0)
    is0 = i < (pl.num_programs(0) // 2)
    f = feats_ref[...]
    cand = cand_ref[0]
    vals = values_ref[0]
    c0v = cand[:, 0]
    best = jnp.abs(f - c0v)
    q = jnp.broadcast_to(c0v, f.shape)
    for k in range(1, NCAND):
        ck = cand[:, k]
        d = jnp.abs(f - ck)
        better = d < best
        q = jnp.where(better, ck, q)
        best = jnp.where(better, d, best)
    w1 = vals[1] - vals[3]
    w2 = vals[2] - vals[3]
    w3 = vals[0] - vals[1] - vals[2] + vals[3]
    wa = jnp.where(is0, w1, w2)
    a0 = jnp.sum(q[:, :H] * wa[:H], axis=1)
    a1 = jnp.sum(q[:, H:] * wa[H:], axis=1)
    m_ref[...] = q * jnp.where(is0, w3, jnp.ones_like(w3))
    aux_ref[...] = jnp.stack([a0, a1], axis=0)
    base = vals[3]
    c0 = jnp.sum(base[:H])
    c1 = jnp.sum(base[H:])
    esc = jnp.exp(scale_ref[0])
    b = bias_ref[0]
    lane16 = lax.broadcasted_iota(jnp.int32, (1, L), 1)
    params = jnp.where(lane16 == 0, esc, jnp.zeros((1, L), jnp.float32))
    params = jnp.where(lane16 == 1, b, params)
    params = jnp.where(lane16 == 2, c0, params)
    params = jnp.where(lane16 == 3, c1, params)
    params_ref[...] = params


def _prep(feats, candidates, values, scale, bias):
    grid = SUMD // RB
    return pl.pallas_call(
        _prep_body,
        grid=(grid,),
        in_specs=[
            pl.BlockSpec((RB, F), lambda i: (i, 0)),
            pl.BlockSpec((1, F, NCAND), lambda i: (i // (SUMD // RB // 2), 0, 0)),
            pl.BlockSpec((1, 4, F), lambda i: (0, 0, 0)),
            pl.BlockSpec(memory_space=pltpu.SMEM),
            pl.BlockSpec(memory_space=pltpu.SMEM),
        ],
        out_specs=[
            pl.BlockSpec((RB, F), lambda i: (i, 0)),
            pl.BlockSpec((2, RB), lambda i: (0, i)),
            pl.BlockSpec((1, L), lambda i: (0, 0)),
        ],
        out_shape=[
            jax.ShapeDtypeStruct((SUMD, F), jnp.float32),
            jax.ShapeDtypeStruct((2, SUMD), jnp.float32),
            jax.ShapeDtypeStruct((1, L), jnp.float32),
        ],
    )(feats, candidates, values, scale, bias)


def _sc_body(m_hbm, aux_hbm, idx_hbm, params_hbm, out_hbm,
             aux0_v, aux1_v, params_v,
             ia0_v, ia1_v, ib0_v, ib1_v,
             ra0_v, ra1_v, rb0_v, rb1_v,
             s0_v, p0_v, p1_v,
             semA0, semA1, semB0, semB1):
    wid = lax.axis_index("s") * NC + lax.axis_index("c")
    per_w = BQ // NW
    nchunk = per_w // K
    w_base = wid * per_w
    lanes = lax.broadcasted_iota(jnp.int32, (L,), 0)
    last = nchunk - 1

    pltpu.sync_copy(aux_hbm.at[0], aux0_v)
    pltpu.sync_copy(aux_hbm.at[1], aux1_v)
    pltpu.sync_copy(params_hbm.at[0], params_v)
    escv = plsc.load_gather(params_v, [jnp.full((L,), 0, jnp.int32)])
    biasv = plsc.load_gather(params_v, [jnp.full((L,), 1, jnp.int32)])
    c0v = plsc.load_gather(params_v, [jnp.full((L,), 2, jnp.int32)])
    c1v = plsc.load_gather(params_v, [jnp.full((L,), 3, jnp.int32)])

    def load_idx(ci, i0_v, i1_v):
        base = w_base + ci * K
        pltpu.sync_copy(idx_hbm.at[0, pl.ds(base, K)], i0_v)
        pltpu.sync_copy(idx_hbm.at[1, pl.ds(base, K)], i1_v)
        for t in range(K // L):
            i1_v[pl.ds(t * L, L)] = i1_v[pl.ds(t * L, L)] + D0

    def fire(i0_v, i1_v, r0_v, r1_v, sem0, sem1):
        pltpu.make_async_copy(m_hbm.at[i0_v], r0_v, sem0).start()
        pltpu.make_async_copy(m_hbm.at[i1_v], r1_v, sem1).start()

    def drain(i0_v, i1_v, r0_v, r1_v, sem0, sem1):
        pltpu.make_async_copy(m_hbm.at[i0_v], r0_v, sem0).wait()
        pltpu.make_async_copy(m_hbm.at[i1_v], r1_v, sem1).wait()

    def compute(ci, i0_v, i1_v, r0_v, r1_v):
        def group(gi, carry):
            j0 = gi * L
            for jj in range(L):
                acc0 = r0_v[j0 + jj, pl.ds(0, L)] * r1_v[j0 + jj, pl.ds(0, L)]
                acc1 = r0_v[j0 + jj, pl.ds(8 * L, L)] * r1_v[j0 + jj, pl.ds(8 * L, L)]
                for t in range(1, 8):
                    acc0 = acc0 + r0_v[j0 + jj, pl.ds(t * L, L)] * r1_v[j0 + jj, pl.ds(t * L, L)]
                for t in range(9, 16):
                    acc1 = acc1 + r0_v[j0 + jj, pl.ds(t * L, L)] * r1_v[j0 + jj, pl.ds(t * L, L)]
                p0_v[jj, :] = acc0
                p1_v[jj, :] = acc1
            o0 = jnp.zeros((L,), jnp.float32)
            o1 = jnp.zeros((L,), jnp.float32)
            for l in range(L):
                col = jnp.full((L,), l, jnp.int32)
                o0 = o0 + plsc.load_gather(p0_v, [lanes, col])
                o1 = o1 + plsc.load_gather(p1_v, [lanes, col])
            i0reg = i0_v[pl.ds(j0, L)]
            i1reg = i1_v[pl.ds(j0, L)]
            o0 = o0 + plsc.load_gather(aux0_v, [i0reg]) + plsc.load_gather(aux0_v, [i1reg])
            o1 = o1 + plsc.load_gather(aux1_v, [i0reg]) + plsc.load_gather(aux1_v, [i1reg])
            s0 = o0 + c0v
            s1 = o1 + c1v
            e2 = jnp.exp(s1 + s1)
            th = 1.0 - 2.0 / (e2 + 1.0)
            s0_v[pl.ds(j0, L)] = s0 * th * escv + biasv
            return carry

        lax.fori_loop(0, K // L, group, 0)
        base = w_base + ci * K
        pltpu.sync_copy(s0_v, out_hbm.at[pl.ds(base, K)])

    load_idx(0, ia0_v, ia1_v)
    fire(ia0_v, ia1_v, ra0_v, ra1_v, semA0, semA1)
    load_idx(1, ib0_v, ib1_v)
    fire(ib0_v, ib1_v, rb0_v, rb1_v, semB0, semB1)

    def pair(p, carry):
        c = 2 * p
        drain(ia0_v, ia1_v, ra0_v, ra1_v, semA0, semA1)
        compute(c, ia0_v, ia1_v, ra0_v, ra1_v)
        cn = jnp.minimum(c + 2, last)
        load_idx(cn, ia0_v, ia1_v)
        fire(ia0_v, ia1_v, ra0_v, ra1_v, semA0, semA1)
        drain(ib0_v, ib1_v, rb0_v, rb1_v, semB0, semB1)
        compute(c + 1, ib0_v, ib1_v, rb0_v, rb1_v)
        cn = jnp.minimum(c + 3, last)
        load_idx(cn, ib0_v, ib1_v)
        fire(ib0_v, ib1_v, rb0_v, rb1_v, semB0, semB1)
        return carry

    lax.fori_loop(0, nchunk // 2, pair, 0)
    drain(ia0_v, ia1_v, ra0_v, ra1_v, semA0, semA1)
    drain(ib0_v, ib1_v, rb0_v, rb1_v, semB0, semB1)


def _sc_contract(m, aux, idx, params):
    return pl.kernel(
        _sc_body,
        out_type=jax.ShapeDtypeStruct((BQ,), jnp.float32),
        mesh=plsc.VectorSubcoreMesh(core_axis_name="c", subcore_axis_name="s"),
        compiler_params=pltpu.CompilerParams(needs_layout_passes=False),
        scratch_types=[
            pltpu.VMEM((SUMD,), jnp.float32),
            pltpu.VMEM((SUMD,), jnp.float32),
            pltpu.VMEM((L,), jnp.float32),
            pltpu.VMEM((K,), jnp.int32),
            pltpu.VMEM((K,), jnp.int32),
            pltpu.VMEM((K,), jnp.int32),
            pltpu.VMEM((K,), jnp.int32),
            pltpu.VMEM((K, F), jnp.float32),
            pltpu.VMEM((K, F), jnp.float32),
            pltpu.VMEM((K, F), jnp.float32),
            pltpu.VMEM((K, F), jnp.float32),
            pltpu.VMEM((K,), jnp.float32),
            pltpu.VMEM((L, L), jnp.float32),
            pltpu.VMEM((L, L), jnp.float32),
            pltpu.SemaphoreType.DMA,
            pltpu.SemaphoreType.DMA,
            pltpu.SemaphoreType.DMA,
            pltpu.SemaphoreType.DMA,
        ],
    )(m, aux, idx, params)


def kernel(idxs, values, feats, candidates, scale, bias, which_axis):
    idx = idxs.astype(jnp.int32)
    m, aux, params = _prep(feats, candidates, values, scale, bias)
    return _sc_contract(m, aux, idx, params)
